# baseline (speedup 1.0000x reference)
import functools

import jax
import jax.numpy as jnp
from jax import lax
from jax.experimental import pallas as pl
from jax.experimental.pallas import tpu as pltpu
from jax.experimental.pallas import tpu_sc as plsc

VOCAB = 1_000_000
EMBED = 32
BATCH = 4096
LABELS = 50

NC = 2
NS = 16
NW = NC * NS

LINES = VOCAB // 4

RB = 160
LB = RB // 4
NBLK = VOCAB // RB
BLK_ITERS = -(-NBLK // NW)

ROWS_PER_W = BATCH // NW
IDXW = 100
IDX_ROWS = BATCH * LABELS // IDXW
IDX_ROWS_PER_W = IDX_ROWS // NW
K = 4
SK = 104
CH = K * IDXW
CHUNKS = IDX_ROWS_PER_W // K
BR = CH // LABELS
LPAD = 64

_DIMNUM = lax.GatherDimensionNumbers(
    offset_dims=(), collapsed_slice_dims=(0,), start_index_map=(0,))


def _splat(v, i):
    idx = jnp.full((16, 1), i, dtype=jnp.int32)
    return lax.gather(v, idx, _DIMNUM, slice_sizes=(1,),
                      mode=lax.GatherScatterMode.PROMISE_IN_BOUNDS)


def _wid():
    return lax.axis_index("s") * NC + lax.axis_index("c")


def _compact_body(emb_hbm, out_hbm, in_v, out_v):
    wid = _wid()

    def blk(i, carry):
        b = wid + i * NW

        @pl.when(b < NBLK)
        def _():
            pltpu.sync_copy(emb_hbm.at[pl.ds(b * RB, RB)], in_v)

            def line(j, carry2):
                for k in range(4):
                    out_v[j, 32 * k:32 * k + 16] = in_v[4 * j + k, 0:16]
                    out_v[j, 32 * k + 16:32 * k + 32] = in_v[4 * j + k, 16:32]
                return carry2

            lax.fori_loop(0, LB, line, 0)
            pltpu.sync_copy(out_v, out_hbm.at[pl.ds(b * LB, LB)])

        return carry

    lax.fori_loop(0, BLK_ITERS, blk, 0)


def _gather_body(lines_hbm, idx_hbm, pos_hbm, out_hbm, idx_v, pos_v, buf_v,
                 out_v, sem):
    wid = _wid()
    lane = lax.iota(jnp.int32, 16)
    pltpu.sync_copy(idx_hbm.at[pl.ds(wid * IDX_ROWS_PER_W, IDX_ROWS_PER_W)],
                    idx_v)
    pltpu.sync_copy(pos_hbm.at[pl.ds(wid * ROWS_PER_W, ROWS_PER_W)], pos_v)

    def chunk(c, carry):
        copies = [
            pltpu.async_copy(
                lines_hbm.at[idx_v.at[c * K + j]],
                buf_v.at[pl.ds(j * SK, IDXW)],
                sem,
            )
            for j in range(K)
        ]
        for cp in copies:
            cp.wait()

        def body(r, carry2):
            a0 = jnp.zeros((16,), jnp.float32)
            a1 = jnp.zeros((16,), jnp.float32)
            row = c * BR + r

            def acc(bv, i, a0, a1):
                b = _splat(bv, i)
                rowv = lax.shift_right_logical(b, 7)
                colv = (b & 127) + lane
                a0 = a0 + plsc.load_gather(buf_v, [rowv, colv])
                a1 = a1 + plsc.load_gather(buf_v, [rowv, colv + 16])
                return a0, a1

            for g in range(3):
                bv = pos_v[row, 16 * g:16 * g + 16]
                for i in range(16):
                    a0, a1 = acc(bv, i, a0, a1)
            bv = pos_v[row, 48:64]
            for i in range(2):
                a0, a1 = acc(bv, i, a0, a1)
            out_v[r, 0:16] = a0
            out_v[r, 16:32] = a1
            return carry2

        lax.fori_loop(0, BR, body, 0)
        pltpu.sync_copy(out_v, out_hbm.at[pl.ds(wid * ROWS_PER_W + c * BR, BR)])
        return carry

    lax.fori_loop(0, CHUNKS, chunk, 0)


_MESH = plsc.VectorSubcoreMesh(core_axis_name="c", subcore_axis_name="s")
_PARAMS = pltpu.CompilerParams(needs_layout_passes=False)


@jax.jit
def _run(inputs, emb):
    flat = inputs.reshape(-1).astype(jnp.int32)
    lines_idx = (flat // 4).reshape(IDX_ROWS, IDXW)
    p = jnp.arange(BATCH * LABELS, dtype=jnp.int32) % CH
    pos = (SK * (p // IDXW) + p % IDXW) * 128 + (flat % 4) * 32
    pos = pos.reshape(BATCH, LABELS)
    pos = jnp.concatenate(
        [pos, jnp.zeros((BATCH, LPAD - LABELS), jnp.int32)], axis=1)

    compact = functools.partial(
        pl.kernel,
        mesh=_MESH,
        compiler_params=_PARAMS,
        out_type=jax.ShapeDtypeStruct((LINES, 128), jnp.float32),
        scratch_types=[
            pltpu.VMEM((RB, EMBED), jnp.float32),
            pltpu.VMEM((LB, 128), jnp.float32),
        ],
    )(_compact_body)
    table = compact(emb)

    gather = functools.partial(
        pl.kernel,
        mesh=_MESH,
        compiler_params=_PARAMS,
        out_type=jax.ShapeDtypeStruct((BATCH, EMBED), jnp.float32),
        scratch_types=[
            pltpu.VMEM((IDX_ROWS_PER_W, IDXW), jnp.int32),
            pltpu.VMEM((ROWS_PER_W, LPAD), jnp.int32),
            pltpu.VMEM(((K - 1) * SK + IDXW, 128), jnp.float32),
            pltpu.VMEM((BR, EMBED), jnp.float32),
            pltpu.SemaphoreType.DMA,
        ],
    )(_gather_body)
    return gather(table, lines_idx, pos)


def kernel(inputs, emb):
    return _run(inputs, emb)


# speedup vs baseline: 1.1418x; 1.1418x over previous
import functools

import jax
import jax.numpy as jnp
from jax import lax
from jax.experimental import pallas as pl
from jax.experimental.pallas import tpu as pltpu
from jax.experimental.pallas import tpu_sc as plsc

VOCAB = 1_000_000
EMBED = 32
BATCH = 4096
LABELS = 50

NC = 2
NS = 16
NW = NC * NS

LINES = VOCAB // 4

TW = 512
NFULL = VOCAB // TW
TAIL0 = NFULL * TW
TAILW = VOCAB - TAIL0
A_STEPS = -(-NFULL // NW) // 2
TAIL_WID = NW - 1

ROWS_PER_W = BATCH // NW
IDXW = LABELS
IDX_ROWS = BATCH
IDX_ROWS_PER_W = IDX_ROWS // NW
K = 4
SK = 56
CH = K * IDXW
CHUNKS = IDX_ROWS_PER_W // K
BR = CH // LABELS
LPAD = 64

_DIMNUM = lax.GatherDimensionNumbers(
    offset_dims=(), collapsed_slice_dims=(0,), start_index_map=(0,))


def _splat(v, i):
    idx = jnp.full((16, 1), i, dtype=jnp.int32)
    return lax.gather(v, idx, _DIMNUM, slice_sizes=(1,),
                      mode=lax.GatherScatterMode.PROMISE_IN_BOUNDS)


def _wid():
    return lax.axis_index("s") * NC + lax.axis_index("c")


def _transpose_block(lane, in_v, out_v, ncols):
    def line(j, carry):
        for k in range(4):
            colv = jnp.full((16,), 4 * j + k, dtype=jnp.int32)
            out_v[j, 32 * k:32 * k + 16] = plsc.load_gather(in_v, [lane, colv])
            out_v[j, 32 * k + 16:32 * k + 32] = plsc.load_gather(
                in_v, [lane + 16, colv])
        return carry

    lax.fori_loop(0, ncols // 4, line, 0)


def _tr_body(embT_hbm, out_hbm, in0, in1, out0, out1, tail_v, rs0, rs1, ws0,
             ws1):
    wid = _wid()
    lane = lax.iota(jnp.int32, 16)
    ins, outs, rs, ws = (in0, in1), (out0, out1), (rs0, rs1), (ws0, ws1)

    for p in range(2):
        b = wid + p * NW
        pltpu.async_copy(embT_hbm.at[:, pl.ds(b * TW, TW)], ins[p], rs[p])

    def step(s, carry):
        for p in range(2):
            i = 2 * s + p
            b = wid + i * NW
            b2 = wid + (i + 2) * NW

            @pl.when(b < NFULL)
            def _():
                pltpu.make_async_copy(
                    embT_hbm.at[:, pl.ds(b * TW, TW)], ins[p], rs[p]).wait()

                @pl.when(i >= 2)
                def _():
                    pltpu.make_async_copy(
                        outs[p], out_hbm.at[pl.ds(0, TW // 4)], ws[p]).wait()

                _transpose_block(lane, ins[p], outs[p], TW)
                pltpu.async_copy(
                    outs[p], out_hbm.at[pl.ds(b * (TW // 4), TW // 4)], ws[p])

                @pl.when(b2 < NFULL)
                def _():
                    pltpu.async_copy(
                        embT_hbm.at[:, pl.ds(b2 * TW, TW)], ins[p], rs[p])

        return carry

    lax.fori_loop(0, A_STEPS, step, 0)

    pltpu.make_async_copy(outs[0], out_hbm.at[pl.ds(0, TW // 4)], ws[0]).wait()
    pltpu.make_async_copy(outs[1], out_hbm.at[pl.ds(0, TW // 4)], ws[1]).wait()

    @pl.when(wid == TAIL_WID)
    def _():
        pltpu.sync_copy(embT_hbm.at[:, pl.ds(TAIL0, TAILW)], tail_v)
        _transpose_block(lane, tail_v, outs[1], TAILW)
        pltpu.sync_copy(outs[1].at[pl.ds(0, TAILW // 4)],
                        out_hbm.at[pl.ds(TAIL0 // 4, TAILW // 4)])


def _gather_body(lines_hbm, idx_hbm, pos_hbm, outT_hbm, idx_v, pos_v, buf0,
                 buf1, out_v, gs0, gs1):
    wid = _wid()
    lane = lax.iota(jnp.int32, 16)
    bufs, gs = (buf0, buf1), (gs0, gs1)
    pltpu.sync_copy(idx_hbm.at[pl.ds(wid * IDX_ROWS_PER_W, IDX_ROWS_PER_W)],
                    idx_v)
    pltpu.sync_copy(pos_hbm.at[pl.ds(wid * ROWS_PER_W, ROWS_PER_W)], pos_v)

    def fire(c, buf, sem):
        for j in range(K):
            pltpu.async_copy(
                lines_hbm.at[idx_v.at[c * K + j]],
                buf.at[pl.ds(j * SK, IDXW)], sem)

    def drain(c, buf, sem):
        for j in range(K):
            pltpu.make_async_copy(
                lines_hbm.at[idx_v.at[c * K + j]],
                buf.at[pl.ds(j * SK, IDXW)], sem).wait()

    fire(0, bufs[0], gs[0])

    def step(s, carry):
        for p in range(2):
            c = 2 * s + p

            @pl.when(c + 1 < CHUNKS)
            def _():
                fire(c + 1, bufs[1 - p], gs[1 - p])

            drain(c, bufs[p], gs[p])
            buf = bufs[p]

            def body(r, carry2):
                a0 = jnp.zeros((16,), jnp.float32)
                a1 = jnp.zeros((16,), jnp.float32)
                row = c * BR + r

                def acc(bv, i, a0, a1):
                    b = _splat(bv, i)
                    rowv = lax.shift_right_logical(b, 7)
                    colv = (b & 127) + lane
                    a0 = a0 + plsc.load_gather(buf, [rowv, colv])
                    a1 = a1 + plsc.load_gather(buf, [rowv, colv + 16])
                    return a0, a1

                for g in range(3):
                    bv = pos_v[row, 16 * g:16 * g + 16]
                    for i in range(16):
                        a0, a1 = acc(bv, i, a0, a1)
                bv = pos_v[row, 48:64]
                for i in range(2):
                    a0, a1 = acc(bv, i, a0, a1)
                colv = jnp.full((16,), row, dtype=jnp.int32)
                plsc.store_scatter(out_v, [lane, colv], a0)
                plsc.store_scatter(out_v, [lane + 16, colv], a1)
                return carry2

            lax.fori_loop(0, BR, body, 0)

        return carry

    lax.fori_loop(0, CHUNKS // 2, step, 0)
    pltpu.sync_copy(out_v, outT_hbm.at[:, pl.ds(wid * ROWS_PER_W, ROWS_PER_W)])


_MESH = plsc.VectorSubcoreMesh(core_axis_name="c", subcore_axis_name="s")
_PARAMS = pltpu.CompilerParams(needs_layout_passes=False)


@jax.jit
def _run(inputs, emb):
    embT = emb.T
    flat = inputs.reshape(-1).astype(jnp.int32)
    lines_idx = (flat // 4).reshape(IDX_ROWS, IDXW)
    p = jnp.arange(BATCH * LABELS, dtype=jnp.int32) % CH
    pos = (SK * (p // IDXW) + p % IDXW) * 128 + (flat % 4) * 32
    pos = pos.reshape(BATCH, LABELS)
    pos = jnp.concatenate(
        [pos, jnp.zeros((BATCH, LPAD - LABELS), jnp.int32)], axis=1)

    transpose = functools.partial(
        pl.kernel,
        mesh=_MESH,
        compiler_params=_PARAMS,
        out_type=jax.ShapeDtypeStruct((LINES, 128), jnp.float32),
        scratch_types=[
            pltpu.VMEM((EMBED, TW), jnp.float32),
            pltpu.VMEM((EMBED, TW), jnp.float32),
            pltpu.VMEM((TW // 4, 128), jnp.float32),
            pltpu.VMEM((TW // 4, 128), jnp.float32),
            pltpu.VMEM((EMBED, TAILW), jnp.float32),
            pltpu.SemaphoreType.DMA,
            pltpu.SemaphoreType.DMA,
            pltpu.SemaphoreType.DMA,
            pltpu.SemaphoreType.DMA,
        ],
    )(_tr_body)
    table = transpose(embT)

    gather = functools.partial(
        pl.kernel,
        mesh=_MESH,
        compiler_params=_PARAMS,
        out_type=jax.ShapeDtypeStruct((EMBED, BATCH), jnp.float32),
        scratch_types=[
            pltpu.VMEM((IDX_ROWS_PER_W, IDXW), jnp.int32),
            pltpu.VMEM((ROWS_PER_W, LPAD), jnp.int32),
            pltpu.VMEM(((K - 1) * SK + IDXW, 128), jnp.float32),
            pltpu.VMEM(((K - 1) * SK + IDXW, 128), jnp.float32),
            pltpu.VMEM((EMBED, ROWS_PER_W), jnp.float32),
            pltpu.SemaphoreType.DMA,
            pltpu.SemaphoreType.DMA,
        ],
    )(_gather_body)
    outT = gather(table, lines_idx, pos)
    return outT.T


def kernel(inputs, emb):
    return _run(inputs, emb)


# speedup vs baseline: 1.6473x; 1.4428x over previous
import functools

import jax
import jax.numpy as jnp
from jax import lax
from jax.experimental import pallas as pl
from jax.experimental.pallas import tpu as pltpu
from jax.experimental.pallas import tpu_sc as plsc

VOCAB = 1_000_000
EMBED = 32
BATCH = 4096
LABELS = 50

NC = 2
NS = 16
NW = NC * NS

LINES = VOCAB // 4

TCW = 2048
TCL = TCW // 4
TC_GRID = -(-VOCAB // TCW)

ROWS_PER_W = BATCH // NW
IDXW = LABELS
IDX_ROWS = BATCH
IDX_ROWS_PER_W = IDX_ROWS // NW
K = 4
SK = 56
CH = K * IDXW
CHUNKS = IDX_ROWS_PER_W // K
BR = CH // LABELS
LPAD = 64

_DIMNUM = lax.GatherDimensionNumbers(
    offset_dims=(), collapsed_slice_dims=(0,), start_index_map=(0,))


def _splat(v, i):
    idx = jnp.full((16, 1), i, dtype=jnp.int32)
    return lax.gather(v, idx, _DIMNUM, slice_sizes=(1,),
                      mode=lax.GatherScatterMode.PROMISE_IN_BOUNDS)


def _wid():
    return lax.axis_index("s") * NC + lax.axis_index("c")


def _tc_tr_body(in_ref, out_ref):
    eye = jnp.eye(EMBED, dtype=jnp.float32)
    t = lax.dot_general(in_ref[...], eye, (((0,), (0,)), ((), ())),
                        preferred_element_type=jnp.float32)
    t4 = t.reshape(TCL, 4, EMBED)
    out_ref[...] = jnp.concatenate([t4[:, k, :] for k in range(4)], axis=1)


def _gather_body(lines_hbm, idx_hbm, pos_hbm, outT_hbm, idx_v, pos_v, buf0,
                 buf1, out_v, gs0, gs1):
    wid = _wid()
    lane = lax.iota(jnp.int32, 16)
    bufs, gs = (buf0, buf1), (gs0, gs1)
    pltpu.sync_copy(idx_hbm.at[pl.ds(wid * IDX_ROWS_PER_W, IDX_ROWS_PER_W)],
                    idx_v)
    pltpu.sync_copy(pos_hbm.at[pl.ds(wid * ROWS_PER_W, ROWS_PER_W)], pos_v)

    def fire(c, buf, sem):
        for j in range(K):
            pltpu.async_copy(
                lines_hbm.at[idx_v.at[c * K + j]],
                buf.at[pl.ds(j * SK, IDXW)], sem)

    def drain(c, buf, sem):
        for j in range(K):
            pltpu.make_async_copy(
                lines_hbm.at[idx_v.at[c * K + j]],
                buf.at[pl.ds(j * SK, IDXW)], sem).wait()

    fire(0, bufs[0], gs[0])

    def step(s, carry):
        for p in range(2):
            c = 2 * s + p

            @pl.when(c + 1 < CHUNKS)
            def _():
                fire(c + 1, bufs[1 - p], gs[1 - p])

            drain(c, bufs[p], gs[p])
            buf = bufs[p]

            def body(r, carry2):
                a0 = jnp.zeros((16,), jnp.float32)
                a1 = jnp.zeros((16,), jnp.float32)
                row = c * BR + r

                def acc(bv, i, a0, a1):
                    b = _splat(bv, i)
                    rowv = lax.shift_right_logical(b, 7)
                    colv = (b & 127) + lane
                    a0 = a0 + plsc.load_gather(buf, [rowv, colv])
                    a1 = a1 + plsc.load_gather(buf, [rowv, colv + 16])
                    return a0, a1

                for g in range(3):
                    bv = pos_v[row, 16 * g:16 * g + 16]
                    for i in range(16):
                        a0, a1 = acc(bv, i, a0, a1)
                bv = pos_v[row, 48:64]
                for i in range(2):
                    a0, a1 = acc(bv, i, a0, a1)
                colv = jnp.full((16,), row, dtype=jnp.int32)
                plsc.store_scatter(out_v, [lane, colv], a0)
                plsc.store_scatter(out_v, [lane + 16, colv], a1)
                return carry2

            lax.fori_loop(0, BR, body, 0)

        return carry

    lax.fori_loop(0, CHUNKS // 2, step, 0)
    pltpu.sync_copy(out_v, outT_hbm.at[:, pl.ds(wid * ROWS_PER_W, ROWS_PER_W)])


_MESH = plsc.VectorSubcoreMesh(core_axis_name="c", subcore_axis_name="s")
_PARAMS = pltpu.CompilerParams(needs_layout_passes=False)


@jax.jit
def _run(inputs, emb):
    embT = emb.T
    flat = inputs.reshape(-1).astype(jnp.int32)
    lines_idx = (flat // 4).reshape(IDX_ROWS, IDXW)
    p = jnp.arange(BATCH * LABELS, dtype=jnp.int32) % CH
    pos = (SK * (p // IDXW) + p % IDXW) * 128 + (flat % 4) * 32
    pos = pos.reshape(BATCH, LABELS)
    pos = jnp.concatenate(
        [pos, jnp.zeros((BATCH, LPAD - LABELS), jnp.int32)], axis=1)

    table = pl.pallas_call(
        _tc_tr_body,
        grid=(TC_GRID,),
        in_specs=[pl.BlockSpec((EMBED, TCW), lambda j: (0, j))],
        out_specs=pl.BlockSpec((TCL, 128), lambda j: (j, 0)),
        out_shape=jax.ShapeDtypeStruct((LINES, 128), jnp.float32),
    )(embT)

    gather = functools.partial(
        pl.kernel,
        mesh=_MESH,
        compiler_params=_PARAMS,
        out_type=jax.ShapeDtypeStruct((EMBED, BATCH), jnp.float32),
        scratch_types=[
            pltpu.VMEM((IDX_ROWS_PER_W, IDXW), jnp.int32),
            pltpu.VMEM((ROWS_PER_W, LPAD), jnp.int32),
            pltpu.VMEM(((K - 1) * SK + IDXW, 128), jnp.float32),
            pltpu.VMEM(((K - 1) * SK + IDXW, 128), jnp.float32),
            pltpu.VMEM((EMBED, ROWS_PER_W), jnp.float32),
            pltpu.SemaphoreType.DMA,
            pltpu.SemaphoreType.DMA,
        ],
    )(_gather_body)
    outT = gather(table, lines_idx, pos)
    return outT.T


def kernel(inputs, emb):
    return _run(inputs, emb)


# speedup vs baseline: 1.7878x; 1.0853x over previous
import functools

import jax
import jax.numpy as jnp
from jax import lax
from jax.experimental import pallas as pl
from jax.experimental.pallas import tpu as pltpu
from jax.experimental.pallas import tpu_sc as plsc

VOCAB = 1_000_000
EMBED = 32
BATCH = 4096
LABELS = 50

NC = 2
NS = 16
NW = NC * NS

ROWS_PER_W = BATCH // NW
IDXW = 100
IDX_ROWS = BATCH * LABELS // IDXW
IDX_ROWS_PER_W = IDX_ROWS // NW
K = 16
CHUNKS = IDX_ROWS_PER_W // K
BR_PER_CHUNK = K * IDXW // LABELS


def _sc_body(emb_hbm, idx_hbm, out_hbm, idx_v, rows_v, out_v, sem):
    wid = lax.axis_index("s") * NC + lax.axis_index("c")
    for c in range(CHUNKS):
        row0 = wid * IDX_ROWS_PER_W + c * K
        pltpu.sync_copy(idx_hbm.at[pl.ds(row0, K)], idx_v)
        copies = [
            pltpu.async_copy(
                emb_hbm.at[idx_v.at[j]],
                rows_v.at[pl.ds(j * IDXW, IDXW)],
                sem,
            )
            for j in range(K)
        ]
        for cp in copies:
            cp.wait()

        def body(r, carry):
            base = r * LABELS
            a0 = rows_v[base, 0:16]
            a1 = rows_v[base, 16:32]
            for l in range(1, LABELS):
                a0 = a0 + rows_v[base + l, 0:16]
                a1 = a1 + rows_v[base + l, 16:32]
            out_v[r, 0:16] = a0
            out_v[r, 16:32] = a1
            return carry

        lax.fori_loop(0, BR_PER_CHUNK, body, 0)
        out0 = wid * ROWS_PER_W + c * BR_PER_CHUNK
        pltpu.sync_copy(out_v, out_hbm.at[pl.ds(out0, BR_PER_CHUNK)])


@jax.jit
def _run(inputs, emb):
    idx = inputs.reshape(IDX_ROWS, IDXW).astype(jnp.int32)
    mesh = plsc.VectorSubcoreMesh(core_axis_name="c", subcore_axis_name="s")
    f = functools.partial(
        pl.kernel,
        mesh=mesh,
        compiler_params=pltpu.CompilerParams(use_tc_tiling_on_sc=False),
        out_type=jax.ShapeDtypeStruct((BATCH, EMBED), jnp.float32),
        scratch_types=[
            pltpu.VMEM((K, IDXW), jnp.int32),
            pltpu.VMEM((K * IDXW, EMBED), jnp.float32),
            pltpu.VMEM((BR_PER_CHUNK, EMBED), jnp.float32),
            pltpu.SemaphoreType.DMA,
        ],
    )(_sc_body)
    return f(emb, idx)


def kernel(inputs, emb):
    return _run(inputs, emb)


# speedup vs baseline: 1.8007x; 1.0073x over previous
import functools

import jax
import jax.numpy as jnp
from jax import lax
from jax.experimental import pallas as pl
from jax.experimental.pallas import tpu as pltpu
from jax.experimental.pallas import tpu_sc as plsc

VOCAB = 1_000_000
EMBED = 32
BATCH = 4096
LABELS = 50

NC = 2
NS = 16
NW = NC * NS

ROWS_PER_W = BATCH // NW
IDXW = 128
IDX_ROWS = BATCH * LABELS // IDXW
IDX_ROWS_PER_W = IDX_ROWS // NW
K = 25
CHUNKS = IDX_ROWS_PER_W // K
BR_PER_CHUNK = K * IDXW // LABELS


def _sc_body(emb_hbm, idx_hbm, out_hbm, idx_v, rows_v, out_v, sem):
    wid = lax.axis_index("s") * NC + lax.axis_index("c")
    for c in range(CHUNKS):
        row0 = wid * IDX_ROWS_PER_W + c * K
        pltpu.sync_copy(idx_hbm.at[pl.ds(row0, K)], idx_v)
        copies = [
            pltpu.async_copy(
                emb_hbm.at[idx_v.at[j]],
                rows_v.at[pl.ds(j * IDXW, IDXW)],
                sem,
            )
            for j in range(K)
        ]
        for cp in copies:
            cp.wait()

        def body(r, carry):
            base = r * LABELS
            a0 = rows_v[base, 0:16]
            a1 = rows_v[base, 16:32]
            for l in range(1, LABELS):
                a0 = a0 + rows_v[base + l, 0:16]
                a1 = a1 + rows_v[base + l, 16:32]
            out_v[r, 0:16] = a0
            out_v[r, 16:32] = a1
            return carry

        lax.fori_loop(0, BR_PER_CHUNK, body, 0)
        out0 = wid * ROWS_PER_W + c * BR_PER_CHUNK
        pltpu.sync_copy(out_v, out_hbm.at[pl.ds(out0, BR_PER_CHUNK)])


@jax.jit
def _run(inputs, emb):
    idx = inputs.reshape(IDX_ROWS, IDXW).astype(jnp.int32)
    mesh = plsc.VectorSubcoreMesh(core_axis_name="c", subcore_axis_name="s")
    f = functools.partial(
        pl.kernel,
        mesh=mesh,
        compiler_params=pltpu.CompilerParams(use_tc_tiling_on_sc=False),
        out_type=jax.ShapeDtypeStruct((BATCH, EMBED), jnp.float32),
        scratch_types=[
            pltpu.VMEM((K, IDXW), jnp.int32),
            pltpu.VMEM((K * IDXW, EMBED), jnp.float32),
            pltpu.VMEM((BR_PER_CHUNK, EMBED), jnp.float32),
            pltpu.SemaphoreType.DMA,
        ],
    )(_sc_body)
    return f(emb, idx)


def kernel(inputs, emb):
    return _run(inputs, emb)


# speedup vs baseline: 4.2629x; 2.3673x over previous
import functools

import jax
import jax.numpy as jnp
from jax import lax
from jax.experimental import pallas as pl
from jax.experimental.pallas import tpu as pltpu
from jax.experimental.pallas import tpu_sc as plsc

VOCAB = 1_000_000
EMBED = 32
BATCH = 4096
LABELS = 50

NC = 2
NS = 16
NW = NC * NS

LINES = VOCAB // 4

TW = 512
NFULL = VOCAB // TW
TAIL0 = NFULL * TW
TAILW = VOCAB - TAIL0
A_STEPS = -(-NFULL // NW) // 2
TAIL_WID = NW - 1

ROWS_PER_W = BATCH // NW
IDXW = LABELS
IDX_ROWS = BATCH
IDX_ROWS_PER_W = IDX_ROWS // NW
K = 4
SK = 56
CH = K * IDXW
CHUNKS = IDX_ROWS_PER_W // K
BR = CH // LABELS
LPAD = 64

_DIMNUM = lax.GatherDimensionNumbers(
    offset_dims=(), collapsed_slice_dims=(0,), start_index_map=(0,))


def _splat(v, i):
    idx = jnp.full((16, 1), i, dtype=jnp.int32)
    return lax.gather(v, idx, _DIMNUM, slice_sizes=(1,),
                      mode=lax.GatherScatterMode.PROMISE_IN_BOUNDS)


def _wid():
    return lax.axis_index("s") * NC + lax.axis_index("c")


def _transpose_block(lane, in_v, out_v, ncols):
    msk = ncols - 1

    def col(c0, carry):
        cb = (c0 + 9 * lane) & msk
        line = lax.shift_right_logical(cb, 2)
        word = (cb & 3) * 32
        v0 = plsc.load_gather(in_v, [lane, cb])
        v1 = plsc.load_gather(in_v, [lane + 16, cb])
        plsc.store_scatter(out_v, [line, word + lane], v0)
        plsc.store_scatter(out_v, [line, word + 16 + lane], v1)
        return carry

    lax.fori_loop(0, ncols, col, 0)


def _tr_body(embT_hbm, out_hbm, in0, in1, out0, out1, tail_v, rs0, rs1, ws0,
             ws1):
    wid = _wid()
    lane = lax.iota(jnp.int32, 16)
    ins, outs, rs, ws = (in0, in1), (out0, out1), (rs0, rs1), (ws0, ws1)

    for p in range(2):
        b = wid + p * NW
        pltpu.async_copy(embT_hbm.at[:, pl.ds(b * TW, TW)], ins[p], rs[p])

    def step(s, carry):
        for p in range(2):
            i = 2 * s + p
            b = wid + i * NW
            b2 = wid + (i + 2) * NW

            @pl.when(b < NFULL)
            def _():
                pltpu.make_async_copy(
                    embT_hbm.at[:, pl.ds(b * TW, TW)], ins[p], rs[p]).wait()

                @pl.when(i >= 2)
                def _():
                    pltpu.make_async_copy(
                        outs[p], out_hbm.at[pl.ds(0, TW // 4)], ws[p]).wait()

                _transpose_block(lane, ins[p], outs[p], TW)
                pltpu.async_copy(
                    outs[p], out_hbm.at[pl.ds(b * (TW // 4), TW // 4)], ws[p])

                @pl.when(b2 < NFULL)
                def _():
                    pltpu.async_copy(
                        embT_hbm.at[:, pl.ds(b2 * TW, TW)], ins[p], rs[p])

        return carry

    lax.fori_loop(0, A_STEPS, step, 0)

    pltpu.make_async_copy(outs[0], out_hbm.at[pl.ds(0, TW // 4)], ws[0]).wait()
    pltpu.make_async_copy(outs[1], out_hbm.at[pl.ds(0, TW // 4)], ws[1]).wait()

    @pl.when(wid == TAIL_WID)
    def _():
        pltpu.sync_copy(embT_hbm.at[:, pl.ds(TAIL0, TAILW)], tail_v)
        _transpose_block(lane, tail_v, outs[1], TAILW)
        pltpu.sync_copy(outs[1].at[pl.ds(0, TAILW // 4)],
                        out_hbm.at[pl.ds(TAIL0 // 4, TAILW // 4)])


def _gather_body(lines_hbm, idx_hbm, pos_hbm, outT_hbm, idx_v, pos_v, buf0,
                 buf1, out_v, gs0, gs1):
    wid = _wid()
    lane = lax.iota(jnp.int32, 16)
    bufs, gs = (buf0, buf1), (gs0, gs1)
    pltpu.sync_copy(idx_hbm.at[pl.ds(wid * IDX_ROWS_PER_W, IDX_ROWS_PER_W)],
                    idx_v)
    pltpu.sync_copy(pos_hbm.at[pl.ds(wid * ROWS_PER_W, ROWS_PER_W)], pos_v)

    def fire(c, buf, sem):
        for j in range(K):
            pltpu.async_copy(
                lines_hbm.at[idx_v.at[c * K + j]],
                buf.at[pl.ds(j * SK, IDXW)], sem)

    def drain(c, buf, sem):
        for j in range(K):
            pltpu.make_async_copy(
                lines_hbm.at[idx_v.at[c * K + j]],
                buf.at[pl.ds(j * SK, IDXW)], sem).wait()

    fire(0, bufs[0], gs[0])

    def step(s, carry):
        for p in range(2):
            c = 2 * s + p

            @pl.when(c + 1 < CHUNKS)
            def _():
                fire(c + 1, bufs[1 - p], gs[1 - p])

            drain(c, bufs[p], gs[p])
            buf = bufs[p]

            def body(r, carry2):
                a0 = jnp.zeros((16,), jnp.float32)
                a1 = jnp.zeros((16,), jnp.float32)
                row = c * BR + r

                def acc(bv, i, a0, a1):
                    b = _splat(bv, i)
                    rowv = lax.shift_right_logical(b, 7)
                    colv = (b & 127) + lane
                    a0 = a0 + plsc.load_gather(buf, [rowv, colv])
                    a1 = a1 + plsc.load_gather(buf, [rowv, colv + 16])
                    return a0, a1

                for g in range(3):
                    bv = pos_v[row, 16 * g:16 * g + 16]
                    for i in range(16):
                        a0, a1 = acc(bv, i, a0, a1)
                bv = pos_v[row, 48:64]
                for i in range(2):
                    a0, a1 = acc(bv, i, a0, a1)
                colv = jnp.full((16,), row, dtype=jnp.int32)
                plsc.store_scatter(out_v, [lane, colv], a0)
                plsc.store_scatter(out_v, [lane + 16, colv], a1)
                return carry2

            lax.fori_loop(0, BR, body, 0)

        return carry

    lax.fori_loop(0, CHUNKS // 2, step, 0)
    pltpu.sync_copy(out_v, outT_hbm.at[:, pl.ds(wid * ROWS_PER_W, ROWS_PER_W)])


_MESH = plsc.VectorSubcoreMesh(core_axis_name="c", subcore_axis_name="s")
_PARAMS = pltpu.CompilerParams(needs_layout_passes=False)


@jax.jit
def _run(inputs, emb):
    embT = emb.T
    flat = inputs.reshape(-1).astype(jnp.int32)
    lines_idx = (flat // 4).reshape(IDX_ROWS, IDXW)
    p = jnp.arange(BATCH * LABELS, dtype=jnp.int32) % CH
    pos = (SK * (p // IDXW) + p % IDXW) * 128 + (flat % 4) * 32
    pos = pos.reshape(BATCH, LABELS)
    pos = jnp.concatenate(
        [pos, jnp.zeros((BATCH, LPAD - LABELS), jnp.int32)], axis=1)

    transpose = functools.partial(
        pl.kernel,
        mesh=_MESH,
        compiler_params=_PARAMS,
        out_type=jax.ShapeDtypeStruct((LINES, 128), jnp.float32),
        scratch_types=[
            pltpu.VMEM((EMBED, TW), jnp.float32),
            pltpu.VMEM((EMBED, TW), jnp.float32),
            pltpu.VMEM((TW // 4, 128), jnp.float32),
            pltpu.VMEM((TW // 4, 128), jnp.float32),
            pltpu.VMEM((EMBED, TAILW), jnp.float32),
            pltpu.SemaphoreType.DMA,
            pltpu.SemaphoreType.DMA,
            pltpu.SemaphoreType.DMA,
            pltpu.SemaphoreType.DMA,
        ],
    )(_tr_body)
    table = transpose(embT)

    gather = functools.partial(
        pl.kernel,
        mesh=_MESH,
        compiler_params=_PARAMS,
        out_type=jax.ShapeDtypeStruct((EMBED, BATCH), jnp.float32),
        scratch_types=[
            pltpu.VMEM((IDX_ROWS_PER_W, IDXW), jnp.int32),
            pltpu.VMEM((ROWS_PER_W, LPAD), jnp.int32),
            pltpu.VMEM(((K - 1) * SK + IDXW, 128), jnp.float32),
            pltpu.VMEM(((K - 1) * SK + IDXW, 128), jnp.float32),
            pltpu.VMEM((EMBED, ROWS_PER_W), jnp.float32),
            pltpu.SemaphoreType.DMA,
            pltpu.SemaphoreType.DMA,
        ],
    )(_gather_body)
    outT = gather(table, lines_idx, pos)
    return outT.T


def kernel(inputs, emb):
    return _run(inputs, emb)


# speedup vs baseline: 4.2694x; 1.0015x over previous
import functools

import jax
import jax.numpy as jnp
from jax import lax
from jax.experimental import pallas as pl
from jax.experimental.pallas import tpu as pltpu
from jax.experimental.pallas import tpu_sc as plsc

VOCAB = 1_000_000
EMBED = 32
BATCH = 4096
LABELS = 50

NC = 2
NS = 16
NW = NC * NS

LINES = VOCAB // 4

TW = 512
NFULL = VOCAB // TW
TAIL0 = NFULL * TW
TAILW = VOCAB - TAIL0
A_STEPS = -(-NFULL // NW) // 2
TAIL_WID = NW - 1

ROWS_PER_W = BATCH // NW
IDXW = LABELS
IDX_ROWS = BATCH
IDX_ROWS_PER_W = IDX_ROWS // NW
K = 4
SK = 56
CH = K * IDXW
CHUNKS = IDX_ROWS_PER_W // K
BR = CH // LABELS
LPAD = 64

_DIMNUM = lax.GatherDimensionNumbers(
    offset_dims=(), collapsed_slice_dims=(0,), start_index_map=(0,))


def _splat(v, i):
    idx = jnp.full((16, 1), i, dtype=jnp.int32)
    return lax.gather(v, idx, _DIMNUM, slice_sizes=(1,),
                      mode=lax.GatherScatterMode.PROMISE_IN_BOUNDS)


def _wid():
    return lax.axis_index("s") * NC + lax.axis_index("c")


def _transpose_block(lane, in_v, out_v, ncols):
    msk = ncols - 1

    def col(c0, carry):
        cb = (c0 + 9 * lane) & msk
        line = lax.shift_right_logical(cb, 2)
        word = (cb & 3) * 32 + (line & 7) * 8
        v0 = plsc.load_gather(in_v, [lane, cb])
        v1 = plsc.load_gather(in_v, [lane + 16, cb])
        plsc.store_scatter(out_v, [line, (word + lane) & 127], v0)
        plsc.store_scatter(out_v, [line, (word + 16 + lane) & 127], v1)
        return carry

    lax.fori_loop(0, ncols, col, 0)


def _tr_body(embT_hbm, out_hbm, in0, in1, out0, out1, tail_v, rs0, rs1, ws0,
             ws1):
    wid = _wid()
    lane = lax.iota(jnp.int32, 16)
    ins, outs, rs, ws = (in0, in1), (out0, out1), (rs0, rs1), (ws0, ws1)

    for p in range(2):
        b = wid + p * NW
        pltpu.async_copy(embT_hbm.at[:, pl.ds(b * TW, TW)], ins[p], rs[p])

    def step(s, carry):
        for p in range(2):
            i = 2 * s + p
            b = wid + i * NW
            b2 = wid + (i + 2) * NW

            @pl.when(b < NFULL)
            def _():
                pltpu.make_async_copy(
                    embT_hbm.at[:, pl.ds(b * TW, TW)], ins[p], rs[p]).wait()

                @pl.when(i >= 2)
                def _():
                    pltpu.make_async_copy(
                        outs[p], out_hbm.at[pl.ds(0, TW // 4)], ws[p]).wait()

                _transpose_block(lane, ins[p], outs[p], TW)
                pltpu.async_copy(
                    outs[p], out_hbm.at[pl.ds(b * (TW // 4), TW // 4)], ws[p])

                @pl.when(b2 < NFULL)
                def _():
                    pltpu.async_copy(
                        embT_hbm.at[:, pl.ds(b2 * TW, TW)], ins[p], rs[p])

        return carry

    lax.fori_loop(0, A_STEPS, step, 0)

    pltpu.make_async_copy(outs[0], out_hbm.at[pl.ds(0, TW // 4)], ws[0]).wait()
    pltpu.make_async_copy(outs[1], out_hbm.at[pl.ds(0, TW // 4)], ws[1]).wait()

    @pl.when(wid == TAIL_WID)
    def _():
        pltpu.sync_copy(embT_hbm.at[:, pl.ds(TAIL0, TAILW)], tail_v)
        _transpose_block(lane, tail_v, outs[1], TAILW)
        pltpu.sync_copy(outs[1].at[pl.ds(0, TAILW // 4)],
                        out_hbm.at[pl.ds(TAIL0 // 4, TAILW // 4)])


def _gather_body(lines_hbm, idx_hbm, pos_hbm, outT_hbm, idx_v, pos_v, buf0,
                 buf1, out_v, gs0, gs1):
    wid = _wid()
    lane = lax.iota(jnp.int32, 16)
    bufs, gs = (buf0, buf1), (gs0, gs1)
    pltpu.sync_copy(idx_hbm.at[pl.ds(wid * IDX_ROWS_PER_W, IDX_ROWS_PER_W)],
                    idx_v)
    pltpu.sync_copy(pos_hbm.at[pl.ds(wid * ROWS_PER_W, ROWS_PER_W)], pos_v)

    def fire(c, buf, sem):
        for j in range(K):
            pltpu.async_copy(
                lines_hbm.at[idx_v.at[c * K + j]],
                buf.at[pl.ds(j * SK, IDXW)], sem)

    def drain(c, buf, sem):
        for j in range(K):
            pltpu.make_async_copy(
                lines_hbm.at[idx_v.at[c * K + j]],
                buf.at[pl.ds(j * SK, IDXW)], sem).wait()

    fire(0, bufs[0], gs[0])

    def step(s, carry):
        for p in range(2):
            c = 2 * s + p

            @pl.when(c + 1 < CHUNKS)
            def _():
                fire(c + 1, bufs[1 - p], gs[1 - p])

            drain(c, bufs[p], gs[p])
            buf = bufs[p]

            def body(r, carry2):
                a0 = jnp.zeros((16,), jnp.float32)
                a1 = jnp.zeros((16,), jnp.float32)
                row = c * BR + r

                def acc(bv, i, a0, a1):
                    b = _splat(bv, i)
                    rowv = lax.shift_right_logical(b, 7)
                    base = b & 127
                    a0 = a0 + plsc.load_gather(buf, [rowv, (base + lane) & 127])
                    a1 = a1 + plsc.load_gather(
                        buf, [rowv, (base + 16 + lane) & 127])
                    return a0, a1

                for g in range(3):
                    bv = pos_v[row, 16 * g:16 * g + 16]
                    for i in range(16):
                        a0, a1 = acc(bv, i, a0, a1)
                bv = pos_v[row, 48:64]
                for i in range(2):
                    a0, a1 = acc(bv, i, a0, a1)
                colv = jnp.full((16,), row, dtype=jnp.int32)
                plsc.store_scatter(out_v, [lane, colv], a0)
                plsc.store_scatter(out_v, [lane + 16, colv], a1)
                return carry2

            lax.fori_loop(0, BR, body, 0)

        return carry

    lax.fori_loop(0, CHUNKS // 2, step, 0)
    pltpu.sync_copy(out_v, outT_hbm.at[:, pl.ds(wid * ROWS_PER_W, ROWS_PER_W)])


_MESH = plsc.VectorSubcoreMesh(core_axis_name="c", subcore_axis_name="s")
_PARAMS = pltpu.CompilerParams(needs_layout_passes=False)


@jax.jit
def _run(inputs, emb):
    embT = emb.T
    flat = inputs.reshape(-1).astype(jnp.int32)
    lines_idx = (flat // 4).reshape(IDX_ROWS, IDXW)
    p = jnp.arange(BATCH * LABELS, dtype=jnp.int32) % CH
    line = flat // 4
    pos = (SK * (p // IDXW) + p % IDXW) * 128 \
        + ((flat % 4) * 32 + (line % 8) * 8) % 128
    pos = pos.reshape(BATCH, LABELS)
    pos = jnp.concatenate(
        [pos, jnp.zeros((BATCH, LPAD - LABELS), jnp.int32)], axis=1)

    transpose = functools.partial(
        pl.kernel,
        mesh=_MESH,
        compiler_params=_PARAMS,
        out_type=jax.ShapeDtypeStruct((LINES, 128), jnp.float32),
        scratch_types=[
            pltpu.VMEM((EMBED, TW), jnp.float32),
            pltpu.VMEM((EMBED, TW), jnp.float32),
            pltpu.VMEM((TW // 4, 128), jnp.float32),
            pltpu.VMEM((TW // 4, 128), jnp.float32),
            pltpu.VMEM((EMBED, TAILW), jnp.float32),
            pltpu.SemaphoreType.DMA,
            pltpu.SemaphoreType.DMA,
            pltpu.SemaphoreType.DMA,
            pltpu.SemaphoreType.DMA,
        ],
    )(_tr_body)
    table = transpose(embT)

    gather = functools.partial(
        pl.kernel,
        mesh=_MESH,
        compiler_params=_PARAMS,
        out_type=jax.ShapeDtypeStruct((EMBED, BATCH), jnp.float32),
        scratch_types=[
            pltpu.VMEM((IDX_ROWS_PER_W, IDXW), jnp.int32),
            pltpu.VMEM((ROWS_PER_W, LPAD), jnp.int32),
            pltpu.VMEM(((K - 1) * SK + IDXW, 128), jnp.float32),
            pltpu.VMEM(((K - 1) * SK + IDXW, 128), jnp.float32),
            pltpu.VMEM((EMBED, ROWS_PER_W), jnp.float32),
            pltpu.SemaphoreType.DMA,
            pltpu.SemaphoreType.DMA,
        ],
    )(_gather_body)
    outT = gather(table, lines_idx, pos)
    return outT.T


def kernel(inputs, emb):
    return _run(inputs, emb)
